# baseline (speedup 1.0000x reference)
import functools

import jax
import jax.numpy as jnp
from jax.experimental import pallas as pl
from jax.experimental.pallas import tpu as pltpu

K_NODES = 8192
D_LATENT = 32
ALPHA = 5.0
TILE = 128


def _som_body(z_ref, cb_ref, q1_ref, bmu_ref, zq_ref, csq_ref):
    i = pl.program_id(0)
    cb = cb_ref[...]

    @pl.when(i == 0)
    def _():
        csq_ref[...] = jax.lax.dot_general(
            jnp.ones((1, D_LATENT), jnp.float32), cb * cb,
            (((1,), (1,)), ((), ())),
            preferred_element_type=jnp.float32,
            precision=jax.lax.Precision.HIGHEST)

    z = z_ref[...]
    zsq = jnp.sum(z * z, axis=1, keepdims=True)
    csq = csq_ref[...]
    zc = jax.lax.dot_general(
        z, cb, (((1,), (1,)), ((), ())),
        preferred_element_type=jnp.float32,
        precision=jax.lax.Precision.DEFAULT)
    d = jnp.maximum((zsq + csq) - 2.0 * zc, 0.0)

    m = jnp.min(d, axis=1, keepdims=True)
    lane = jax.lax.broadcasted_iota(jnp.int32, d.shape, 1)
    bmu = jnp.min(jnp.where(d == m, lane, K_NODES), axis=1, keepdims=True)
    bmu_ref[...] = bmu

    onehot = (lane == bmu).astype(jnp.float32)
    cb_hi = cb.astype(jnp.bfloat16).astype(jnp.float32)
    cb_lo = cb - cb_hi
    zq_hi = jax.lax.dot_general(
        onehot, cb_hi, (((1,), (0,)), ((), ())),
        preferred_element_type=jnp.float32,
        precision=jax.lax.Precision.DEFAULT)
    zq_lo = jax.lax.dot_general(
        onehot, cb_lo, (((1,), (0,)), ((), ())),
        preferred_element_type=jnp.float32,
        precision=jax.lax.Precision.DEFAULT)
    zq_ref[...] = zq_hi + zq_lo

    t = 1.0 + d / ALPHA
    r = 1.0 / t
    qu = r * r * r
    s = jnp.sum(qu, axis=1, keepdims=True)
    q = qu * (1.0 / s)
    q1_ref[...] = q


@jax.jit
def kernel(ts_emb_seq, codebook):
    b, t_max, d_latent = ts_emb_seq.shape
    n = b * t_max
    z = ts_emb_seq.reshape(n, d_latent)

    grid = (n // TILE,)
    q1, bmu, zq = pl.pallas_call(
        _som_body,
        grid=grid,
        in_specs=[
            pl.BlockSpec((TILE, D_LATENT), lambda i: (i, 0)),
            pl.BlockSpec((K_NODES, D_LATENT), lambda i: (0, 0)),
        ],
        out_specs=[
            pl.BlockSpec((TILE, K_NODES), lambda i: (i, 0)),
            pl.BlockSpec((TILE, 1), lambda i: (i, 0)),
            pl.BlockSpec((TILE, D_LATENT), lambda i: (i, 0)),
        ],
        out_shape=[
            jax.ShapeDtypeStruct((n, K_NODES), jnp.float32),
            jax.ShapeDtypeStruct((n, 1), jnp.int32),
            jax.ShapeDtypeStruct((n, D_LATENT), jnp.float32),
        ],
        scratch_shapes=[pltpu.VMEM((1, K_NODES), jnp.float32)],
    )(z, codebook)

    return (q1, q1, bmu.reshape(n), zq)


# speedup vs baseline: 1.3186x; 1.3186x over previous
import functools

import jax
import jax.numpy as jnp
from jax import lax
from jax.experimental import pallas as pl
from jax.experimental.pallas import tpu as pltpu
from jax.experimental.pallas import tpu_sc as plsc

K_NODES = 8192
D_LATENT = 32
ALPHA = 5.0
TILE = 128
N_TOKENS = 2048


def _som_body(z_ref, cb_ref, q1_ref, q2_ref, bmu_ref, idx_ref, csq_ref):
    i = pl.program_id(0)
    cb = cb_ref[...]

    @pl.when(i == 0)
    def _():
        csq_ref[...] = jax.lax.dot_general(
            jnp.ones((1, D_LATENT), jnp.float32), cb * cb,
            (((1,), (1,)), ((), ())),
            preferred_element_type=jnp.float32,
            precision=jax.lax.Precision.HIGHEST)

    z = z_ref[...]
    zsq = jnp.sum(z * z, axis=1, keepdims=True)
    csq = csq_ref[...]
    zc = jax.lax.dot_general(
        z, cb, (((1,), (1,)), ((), ())),
        preferred_element_type=jnp.float32,
        precision=jax.lax.Precision.DEFAULT)
    d = jnp.maximum((zsq + csq) - 2.0 * zc, 0.0)

    m = jnp.min(d, axis=1, keepdims=True)
    lane = jax.lax.broadcasted_iota(jnp.int32, d.shape, 1)
    bmu = jnp.min(jnp.where(d == m, lane, K_NODES), axis=1, keepdims=True)
    bmu_ref[...] = bmu

    lane_d = jax.lax.broadcasted_iota(jnp.int32, (TILE, D_LATENT), 1)
    idx_ref[...] = bmu * D_LATENT + lane_d

    t = 1.0 + d / ALPHA
    r = 1.0 / t
    qu = r * r * r
    s = jnp.sum(qu, axis=1, keepdims=True)
    q = qu * (1.0 / s)
    q1_ref[...] = q
    q2_ref[...] = q


def _tc_call(z, codebook):
    n = z.shape[0]
    grid = (n // TILE,)
    return pl.pallas_call(
        _som_body,
        grid=grid,
        in_specs=[
            pl.BlockSpec((TILE, D_LATENT), lambda i: (i, 0)),
            pl.BlockSpec((K_NODES, D_LATENT), lambda i: (0, 0)),
        ],
        out_specs=[
            pl.BlockSpec((TILE, K_NODES), lambda i: (i, 0)),
            pl.BlockSpec((TILE, K_NODES), lambda i: (i, 0)),
            pl.BlockSpec((TILE, 1), lambda i: (i, 0)),
            pl.BlockSpec((TILE, D_LATENT), lambda i: (i, 0)),
        ],
        out_shape=[
            jax.ShapeDtypeStruct((n, K_NODES), jnp.float32),
            jax.ShapeDtypeStruct((n, K_NODES), jnp.float32),
            jax.ShapeDtypeStruct((n, 1), jnp.int32),
            jax.ShapeDtypeStruct((n, D_LATENT), jnp.int32),
        ],
        scratch_shapes=[pltpu.VMEM((1, K_NODES), jnp.float32)],
    )(z, codebook)


def _make_sc_gather(n_elems):
    info = plsc.get_sparse_core_info()
    nc, ns = info.num_cores, info.num_subcores
    nw = nc * ns
    epw = n_elems // nw
    mesh = plsc.VectorSubcoreMesh(core_axis_name="c", subcore_axis_name="s")

    @functools.partial(
        pl.kernel, mesh=mesh,
        out_type=jax.ShapeDtypeStruct((n_elems,), jnp.float32),
        scratch_types=[
            pltpu.VMEM((epw,), jnp.int32),
            pltpu.VMEM((epw,), jnp.float32),
            pltpu.SemaphoreType.DMA,
        ],
    )
    def _gather(table_hbm, idx_hbm, out_hbm, idx_v, vals_v, sem):
        wid = lax.axis_index("s") * nc + lax.axis_index("c")
        base = wid * epw
        pltpu.sync_copy(idx_hbm.at[pl.ds(base, epw)], idx_v)
        pltpu.async_copy(table_hbm.at[idx_v], vals_v, sem).wait()
        pltpu.sync_copy(vals_v, out_hbm.at[pl.ds(base, epw)])

    return _gather


@jax.jit
def kernel(ts_emb_seq, codebook):
    b, t_max, d_latent = ts_emb_seq.shape
    n = b * t_max
    z = ts_emb_seq.reshape(n, d_latent)

    q1, q2, bmu, idx_flat = _tc_call(z, codebook)
    zq_flat = _make_sc_gather(n * d_latent)(
        codebook.reshape(-1), idx_flat.reshape(-1))

    return (q1, q2, bmu.reshape(n), zq_flat.reshape(n, d_latent))


# speedup vs baseline: 1.3707x; 1.0396x over previous
import functools

import jax
import jax.numpy as jnp
from jax import lax
from jax.experimental import pallas as pl
from jax.experimental.pallas import tpu as pltpu
from jax.experimental.pallas import tpu_sc as plsc

K_NODES = 8192
D_LATENT = 32
ALPHA = 5.0
TILE = 256
N_TOKENS = 2048


def _som_body(z_ref, cb_ref, q1_ref, q2_ref, bmu_ref, idx_ref, csq_ref):
    i = pl.program_id(0)
    cb = cb_ref[...]

    @pl.when(i == 0)
    def _():
        csq_ref[...] = jax.lax.dot_general(
            jnp.ones((1, D_LATENT), jnp.float32), cb * cb,
            (((1,), (1,)), ((), ())),
            preferred_element_type=jnp.float32,
            precision=jax.lax.Precision.HIGHEST)

    z = z_ref[...]
    zsq = jnp.sum(z * z, axis=1, keepdims=True)
    csq = csq_ref[...]
    zc = jax.lax.dot_general(
        z, cb, (((1,), (1,)), ((), ())),
        preferred_element_type=jnp.float32,
        precision=jax.lax.Precision.DEFAULT)
    d = jnp.maximum((zsq + csq) - 2.0 * zc, 0.0)

    m = jnp.min(d, axis=1, keepdims=True)
    lane = jax.lax.broadcasted_iota(jnp.int32, d.shape, 1)
    bmu = jnp.min(jnp.where(d == m, lane, K_NODES), axis=1, keepdims=True)
    bmu_ref[...] = bmu

    lane_d = jax.lax.broadcasted_iota(jnp.int32, (TILE, D_LATENT), 1)
    idx_ref[...] = bmu * D_LATENT + lane_d

    t = 1.0 + d / ALPHA
    r = 1.0 / t
    qu = r * r * r
    s = jnp.sum(qu, axis=1, keepdims=True)
    q = qu * (1.0 / s)
    q1_ref[...] = q
    q2_ref[...] = q


def _tc_call(z, codebook):
    n = z.shape[0]
    grid = (n // TILE,)
    return pl.pallas_call(
        _som_body,
        grid=grid,
        in_specs=[
            pl.BlockSpec((TILE, D_LATENT), lambda i: (i, 0)),
            pl.BlockSpec((K_NODES, D_LATENT), lambda i: (0, 0)),
        ],
        out_specs=[
            pl.BlockSpec((TILE, K_NODES), lambda i: (i, 0)),
            pl.BlockSpec((TILE, K_NODES), lambda i: (i, 0)),
            pl.BlockSpec((TILE, 1), lambda i: (i, 0)),
            pl.BlockSpec((TILE, D_LATENT), lambda i: (i, 0)),
        ],
        out_shape=[
            jax.ShapeDtypeStruct((n, K_NODES), jnp.float32),
            jax.ShapeDtypeStruct((n, K_NODES), jnp.float32),
            jax.ShapeDtypeStruct((n, 1), jnp.int32),
            jax.ShapeDtypeStruct((n, D_LATENT), jnp.int32),
        ],
        scratch_shapes=[pltpu.VMEM((1, K_NODES), jnp.float32)],
    )(z, codebook)


def _make_sc_gather(n_elems):
    info = plsc.get_sparse_core_info()
    nc, ns = info.num_cores, info.num_subcores
    nw = nc * ns
    epw = n_elems // nw
    mesh = plsc.VectorSubcoreMesh(core_axis_name="c", subcore_axis_name="s")

    @functools.partial(
        pl.kernel, mesh=mesh,
        out_type=jax.ShapeDtypeStruct((n_elems,), jnp.float32),
        scratch_types=[
            pltpu.VMEM((epw,), jnp.int32),
            pltpu.VMEM((epw,), jnp.float32),
            pltpu.SemaphoreType.DMA,
        ],
    )
    def _gather(table_hbm, idx_hbm, out_hbm, idx_v, vals_v, sem):
        wid = lax.axis_index("s") * nc + lax.axis_index("c")
        base = wid * epw
        pltpu.sync_copy(idx_hbm.at[pl.ds(base, epw)], idx_v)
        pltpu.async_copy(table_hbm.at[idx_v], vals_v, sem).wait()
        pltpu.sync_copy(vals_v, out_hbm.at[pl.ds(base, epw)])

    return _gather


@jax.jit
def kernel(ts_emb_seq, codebook):
    b, t_max, d_latent = ts_emb_seq.shape
    n = b * t_max
    z = ts_emb_seq.reshape(n, d_latent)

    q1, q2, bmu, idx_flat = _tc_call(z, codebook)
    zq_flat = _make_sc_gather(n * d_latent)(
        codebook.reshape(-1), idx_flat.reshape(-1))

    return (q1, q2, bmu.reshape(n), zq_flat.reshape(n, d_latent))
